# baseline (speedup 1.0000x reference)
import functools

import jax
import jax.numpy as jnp
from jax import lax
from jax.experimental import pallas as pl
from jax.experimental.pallas import tpu as pltpu
from jax.experimental.pallas import tpu_sc as plsc

_B = 1024
_L = 200
_H = 64
_BL = _B * _L


def _rsqrt_newton(v):
    i = lax.bitcast_convert_type(v, jnp.int32)
    i = jnp.int32(0x5F3759DF) - lax.shift_right_logical(i, 1)
    y = lax.bitcast_convert_type(i, jnp.float32)
    for _ in range(3):
        y = y * (1.5 - 0.5 * v * y * y)
    return y


def _sc_body(ids_hbm, tt_hbm, word_hbm, pos_hbm, type_hbm, gamma_hbm,
             beta_hbm, out_hbm, idx_v, tidx_v, rows_v, trows_v, pos_v, g_v,
             b_v, sem):
    nc = 2
    wid = lax.axis_index("s") * nc + lax.axis_index("c")
    seq_per_w = _B // 32

    pltpu.sync_copy(pos_hbm.at[pl.ds(0, _L)], pos_v)
    pltpu.sync_copy(gamma_hbm, g_v)
    pltpu.sync_copy(beta_hbm, b_v)

    gam = [g_v[pl.ds(16 * j, 16)] for j in range(4)]
    bet = [b_v[pl.ds(16 * j, 16)] for j in range(4)]

    def seq_body(t, _):
        base = (wid * seq_per_w + t) * _L
        pltpu.sync_copy(ids_hbm.at[pl.ds(base, 104)], idx_v.at[0])
        pltpu.sync_copy(ids_hbm.at[pl.ds(base + 96, 104)], idx_v.at[1])
        pltpu.sync_copy(tt_hbm.at[pl.ds(base, 104)], tidx_v.at[0])
        pltpu.sync_copy(tt_hbm.at[pl.ds(base + 96, 104)], tidx_v.at[1])
        cps = [
            pltpu.make_async_copy(word_hbm.at[idx_v.at[0]],
                                  rows_v.at[pl.ds(0, 104)], sem),
            pltpu.make_async_copy(word_hbm.at[idx_v.at[1]],
                                  rows_v.at[pl.ds(96, 104)], sem),
            pltpu.make_async_copy(type_hbm.at[tidx_v.at[0]],
                                  trows_v.at[pl.ds(0, 104)], sem),
            pltpu.make_async_copy(type_hbm.at[tidx_v.at[1]],
                                  trows_v.at[pl.ds(96, 104)], sem),
        ]
        for cp in cps:
            cp.start()
        for cp in cps:
            cp.wait()

        def row_body(r, _):
            xs = []
            total = jnp.float32(0.0)
            totsq = jnp.float32(0.0)
            for j in range(4):
                x = (rows_v[r, pl.ds(16 * j, 16)]
                     + pos_v[r, pl.ds(16 * j, 16)]
                     + trows_v[r, pl.ds(16 * j, 16)])
                xs.append(x)
                total = total + jnp.sum(x)
                totsq = totsq + jnp.sum(x * x)
            mean = total * jnp.float32(1.0 / _H)
            var = totsq * jnp.float32(1.0 / _H) - mean * mean
            vv = jnp.full((16,), var + jnp.float32(1e-5), jnp.float32)
            scale = _rsqrt_newton(vv)
            meanv = jnp.full((16,), mean, jnp.float32)
            for j in range(4):
                rows_v[r, pl.ds(16 * j, 16)] = (
                    (xs[j] - meanv) * scale * gam[j] + bet[j])
            return ()

        lax.fori_loop(0, _L, row_body, (), unroll=False)
        pltpu.sync_copy(rows_v, out_hbm.at[pl.ds(base, _L)])
        return ()

    lax.fori_loop(0, seq_per_w, seq_body, (), unroll=False)


def kernel(input_ids, token_type_ids, word_emb, pos_emb, type_emb, gamma,
           beta):
    ids_flat = input_ids.reshape(-1).astype(jnp.int32)
    tt_flat = token_type_ids.reshape(-1).astype(jnp.int32)

    mesh = plsc.VectorSubcoreMesh(core_axis_name="c", subcore_axis_name="s")
    run = functools.partial(
        pl.kernel,
        mesh=mesh,
        compiler_params=pltpu.CompilerParams(
            needs_layout_passes=False, use_tc_tiling_on_sc=False),
        out_type=jax.ShapeDtypeStruct((_BL, _H), jnp.float32),
        scratch_types=[
            pltpu.VMEM((2, 104), jnp.int32),
            pltpu.VMEM((2, 104), jnp.int32),
            pltpu.VMEM((_L, _H), jnp.float32),
            pltpu.VMEM((_L, _H), jnp.float32),
            pltpu.VMEM((_L, _H), jnp.float32),
            pltpu.VMEM((_H,), jnp.float32),
            pltpu.VMEM((_H,), jnp.float32),
            pltpu.SemaphoreType.DMA,
        ],
    )(_sc_body)
    out = run(ids_flat, tt_flat, word_emb, pos_emb, type_emb, gamma, beta)
    return out.reshape(_B, _L, _H)


# speedup vs baseline: 4.3527x; 4.3527x over previous
import functools

import jax
import jax.numpy as jnp
from jax import lax
from jax.experimental import pallas as pl
from jax.experimental.pallas import tpu as pltpu
from jax.experimental.pallas import tpu_sc as plsc

_B = 1024
_L = 200
_H = 64
_BL = _B * _L
_NW = 32
_CHUNK = 2 * _L
_NCHUNK = _B // _NW // 2
_OFFS = (0, 96, 200, 296)


def _rsqrt_newton(v):
    i = lax.bitcast_convert_type(v, jnp.int32)
    i = jnp.int32(0x5F3759DF) - lax.shift_right_logical(i, 1)
    y = lax.bitcast_convert_type(i, jnp.float32)
    for _ in range(3):
        y = y * (1.5 - 0.5 * v * y * y)
    return y


def _sc_body(ids_hbm, tt_hbm, word_hbm, pos_hbm, type_hbm, gamma_hbm,
             beta_hbm, out_hbm, idx0, idx1, tt0, tt1, rows0, rows1, comb_v,
             tv_v, g_v, b_v, sem0, sem1):
    wid = lax.axis_index("s") * 2 + lax.axis_index("c")
    wbase = wid * (_NCHUNK * _CHUNK)

    pltpu.sync_copy(pos_hbm.at[pl.ds(0, _L)], comb_v.at[pl.ds(0, _L)])
    pltpu.sync_copy(pos_hbm.at[pl.ds(0, _L)], comb_v.at[pl.ds(_L, _L)])
    pltpu.sync_copy(type_hbm, tv_v)
    pltpu.sync_copy(gamma_hbm, g_v)
    pltpu.sync_copy(beta_hbm, b_v)

    sl = [pl.ds(16 * j, 16) for j in range(4)]
    t0 = [tv_v[0, sl[j]] for j in range(4)]
    t1 = [tv_v[1, sl[j]] for j in range(4)]
    gam = [g_v[sl[j]] for j in range(4)]
    bet = [b_v[sl[j]] for j in range(4)]

    @pl.loop(0, _L)
    def _build(r):
        for j in range(4):
            comb_v[r, sl[j]] = comb_v[r, sl[j]] + t0[j]
            comb_v[_L + r, sl[j]] = comb_v[_L + r, sl[j]] + t1[j]

    bufs = ((idx0, tt0, rows0, sem0), (idx1, tt1, rows1, sem1))

    def issue(c, b):
        idx, tt, rows, sem = bufs[b]
        base = wbase + c * _CHUNK
        for k, off in enumerate(_OFFS):
            pltpu.sync_copy(ids_hbm.at[pl.ds(base + off, 104)], idx.at[k])
        pltpu.sync_copy(tt_hbm.at[pl.ds(base, _CHUNK)], tt)
        for k, off in enumerate(_OFFS):
            pltpu.make_async_copy(word_hbm.at[idx.at[k]],
                                  rows.at[pl.ds(off, 104)], sem).start()

    def process(c, b):
        idx, tt, rows, sem = bufs[b]
        for k, off in enumerate(_OFFS):
            pltpu.make_async_copy(word_hbm.at[idx.at[k]],
                                  rows.at[pl.ds(off, 104)], sem).wait()

        @pl.loop(0, _CHUNK // 16)
        def _group(g):
            rbase = g * 16
            ttv = tt[pl.ds(rbase, 16)]
            for i in range(16):
                r = rbase + i
                cb = ttv[i] * _L + lax.rem(r, _L)
                x = [rows[r, sl[j]] + comb_v[cb, sl[j]] for j in range(4)]
                s = (x[0] + x[1]) + (x[2] + x[3])
                sq = ((x[0] * x[0] + x[1] * x[1])
                      + (x[2] * x[2] + x[3] * x[3]))
                tot = jnp.sum(s)
                tsq = jnp.sum(sq)
                mean = tot * jnp.float32(1.0 / _H)
                var = tsq * jnp.float32(1.0 / _H) - mean * mean
                vv = jnp.full((16,), var + jnp.float32(1e-5), jnp.float32)
                scale = _rsqrt_newton(vv)
                mv = jnp.full((16,), mean, jnp.float32)
                for j in range(4):
                    rows[r, sl[j]] = (x[j] - mv) * scale * gam[j] + bet[j]

        pltpu.sync_copy(rows, out_hbm.at[pl.ds(wbase + c * _CHUNK, _CHUNK)])

    issue(0, 0)

    @pl.loop(0, _NCHUNK // 2)
    def _main(i):
        c0 = i * 2
        issue(c0 + 1, 1)
        process(c0, 0)

        @pl.when(c0 + 2 < _NCHUNK)
        def _():
            issue(c0 + 2, 0)

        process(c0 + 1, 1)


def kernel(input_ids, token_type_ids, word_emb, pos_emb, type_emb, gamma,
           beta):
    ids_flat = input_ids.reshape(-1).astype(jnp.int32)
    tt_flat = token_type_ids.reshape(-1).astype(jnp.int32)

    mesh = plsc.VectorSubcoreMesh(core_axis_name="c", subcore_axis_name="s")
    run = functools.partial(
        pl.kernel,
        mesh=mesh,
        compiler_params=pltpu.CompilerParams(
            needs_layout_passes=False, use_tc_tiling_on_sc=False),
        out_type=jax.ShapeDtypeStruct((_BL, _H), jnp.float32),
        scratch_types=[
            pltpu.VMEM((4, 104), jnp.int32),
            pltpu.VMEM((4, 104), jnp.int32),
            pltpu.VMEM((_CHUNK,), jnp.int32),
            pltpu.VMEM((_CHUNK,), jnp.int32),
            pltpu.VMEM((_CHUNK, _H), jnp.float32),
            pltpu.VMEM((_CHUNK, _H), jnp.float32),
            pltpu.VMEM((2 * _L, _H), jnp.float32),
            pltpu.VMEM((2, _H), jnp.float32),
            pltpu.VMEM((_H,), jnp.float32),
            pltpu.VMEM((_H,), jnp.float32),
            pltpu.SemaphoreType.DMA,
            pltpu.SemaphoreType.DMA,
        ],
    )(_sc_body)
    out = run(ids_flat, tt_flat, word_emb, pos_emb, type_emb, gamma, beta)
    return out.reshape(_B, _L, _H)


# speedup vs baseline: 4.3695x; 1.0039x over previous
import functools

import jax
import jax.numpy as jnp
from jax import lax
from jax.experimental import pallas as pl
from jax.experimental.pallas import tpu as pltpu
from jax.experimental.pallas import tpu_sc as plsc

_B = 1024
_L = 200
_H = 64
_NW = 32
_SEQ_PER_W = _B // _NW
_OFFS = (0, 96)


def _rsqrt_newton(v):
    i = lax.bitcast_convert_type(v, jnp.int32)
    i = jnp.int32(0x5F3759DF) - lax.shift_right_logical(i, 1)
    y = lax.bitcast_convert_type(i, jnp.float32)
    for _ in range(3):
        y = y * (1.5 - 0.5 * v * y * y)
    return y


def _sc_body(ids_hbm, tt_hbm, word_hbm, pos_hbm, type_hbm, gamma_hbm,
             beta_hbm, out_hbm, idx0, idx1, tt0, tt1, rows0, rows1, ob0, ob1,
             comb_v, tv_v, g_v, b_v, gsem0, gsem1, wsem0, wsem1):
    wid = lax.axis_index("s") * 2 + lax.axis_index("c")
    seq0 = wid * _SEQ_PER_W

    pltpu.sync_copy(pos_hbm.at[pl.ds(0, _L)], comb_v.at[pl.ds(0, _L)])
    pltpu.sync_copy(pos_hbm.at[pl.ds(0, _L)], comb_v.at[pl.ds(_L, _L)])
    pltpu.sync_copy(type_hbm, tv_v)
    pltpu.sync_copy(gamma_hbm, g_v)
    pltpu.sync_copy(beta_hbm, b_v)

    sl = [pl.ds(16 * j, 16) for j in range(4)]
    t0 = [tv_v[0, sl[j]] for j in range(4)]
    t1 = [tv_v[1, sl[j]] for j in range(4)]
    gam = [g_v[sl[j]] for j in range(4)]
    bet = [b_v[sl[j]] for j in range(4)]

    @pl.loop(0, _L)
    def _build(r):
        for j in range(4):
            comb_v[r, sl[j]] = comb_v[r, sl[j]] + t0[j]
            comb_v[_L + r, sl[j]] = comb_v[_L + r, sl[j]] + t1[j]

    bufs = ((idx0, tt0, rows0, ob0, gsem0, wsem0),
            (idx1, tt1, rows1, ob1, gsem1, wsem1))

    def wb_copy(c, b):
        ob, wsem = bufs[b][3], bufs[b][5]
        return pltpu.make_async_copy(ob, out_hbm.at[seq0 + c], wsem)

    def issue(c, b):
        idx, tt, rows, _, gsem, _ = bufs[b]
        s = seq0 + c
        for k, off in enumerate(_OFFS):
            pltpu.sync_copy(ids_hbm.at[s, pl.ds(off, 104)], idx.at[k])
        pltpu.sync_copy(tt_hbm.at[s, pl.ds(0, _L)], tt.at[pl.ds(0, _L)])
        for k, off in enumerate(_OFFS):
            pltpu.make_async_copy(word_hbm.at[idx.at[k]],
                                  rows.at[pl.ds(off, 104)], gsem).start()

    def process(c, b):
        idx, tt, rows, ob, gsem, _ = bufs[b]
        for k, off in enumerate(_OFFS):
            pltpu.make_async_copy(word_hbm.at[idx.at[k]],
                                  rows.at[pl.ds(off, 104)], gsem).wait()

        @pl.when(c >= 2)
        def _():
            wb_copy(c - 2, b).wait()

        @pl.loop(0, _L // 8)
        def _group(g):
            rbase = g * 8
            ttv = tt[pl.ds(rbase, 16)]
            for i in range(8):
                r = rbase + i
                cb = ttv[i] * _L + r
                x = [rows[r, sl[j]] + comb_v[cb, sl[j]] for j in range(4)]
                s = (x[0] + x[1]) + (x[2] + x[3])
                sq = ((x[0] * x[0] + x[1] * x[1])
                      + (x[2] * x[2] + x[3] * x[3]))
                tot = jnp.sum(s)
                tsq = jnp.sum(sq)
                mean = tot * jnp.float32(1.0 / _H)
                var = tsq * jnp.float32(1.0 / _H) - mean * mean
                vv = jnp.full((16,), var + jnp.float32(1e-5), jnp.float32)
                scale = _rsqrt_newton(vv)
                mv = jnp.full((16,), mean, jnp.float32)
                for j in range(4):
                    ob[r, sl[j]] = (x[j] - mv) * scale * gam[j] + bet[j]

        wb_copy(c, b).start()

    issue(0, 0)

    @pl.loop(0, _SEQ_PER_W // 2)
    def _main(i):
        c0 = i * 2
        issue(c0 + 1, 1)
        process(c0, 0)

        @pl.when(c0 + 2 < _SEQ_PER_W)
        def _():
            issue(c0 + 2, 0)

        process(c0 + 1, 1)

    wb_copy(_SEQ_PER_W - 2, 0).wait()
    wb_copy(_SEQ_PER_W - 1, 1).wait()


def kernel(input_ids, token_type_ids, word_emb, pos_emb, type_emb, gamma,
           beta):
    mesh = plsc.VectorSubcoreMesh(core_axis_name="c", subcore_axis_name="s")
    run = functools.partial(
        pl.kernel,
        mesh=mesh,
        compiler_params=pltpu.CompilerParams(
            needs_layout_passes=False, use_tc_tiling_on_sc=False),
        out_type=jax.ShapeDtypeStruct((_B, _L, _H), jnp.float32),
        scratch_types=[
            pltpu.VMEM((2, 104), jnp.int32),
            pltpu.VMEM((2, 104), jnp.int32),
            pltpu.VMEM((208,), jnp.int32),
            pltpu.VMEM((208,), jnp.int32),
            pltpu.VMEM((_L, _H), jnp.float32),
            pltpu.VMEM((_L, _H), jnp.float32),
            pltpu.VMEM((_L, _H), jnp.float32),
            pltpu.VMEM((_L, _H), jnp.float32),
            pltpu.VMEM((2 * _L, _H), jnp.float32),
            pltpu.VMEM((2, _H), jnp.float32),
            pltpu.VMEM((_H,), jnp.float32),
            pltpu.VMEM((_H,), jnp.float32),
            pltpu.SemaphoreType.DMA,
            pltpu.SemaphoreType.DMA,
            pltpu.SemaphoreType.DMA,
            pltpu.SemaphoreType.DMA,
        ],
    )(_sc_body)
    return run(input_ids.astype(jnp.int32), token_type_ids.astype(jnp.int32),
               word_emb, pos_emb, type_emb, gamma, beta)


# speedup vs baseline: 5.2707x; 1.2062x over previous
import functools

import jax
import jax.numpy as jnp
from jax import lax
from jax.experimental import pallas as pl
from jax.experimental.pallas import tpu as pltpu
from jax.experimental.pallas import tpu_sc as plsc

_B = 1024
_L = 200
_H = 64
_NW = 32
_SEQ_PER_W = _B // _NW
_OFFS = (0, 96)


def _rsqrt_newton(v):
    i = lax.bitcast_convert_type(v, jnp.int32)
    i = jnp.int32(0x5F3759DF) - lax.shift_right_logical(i, 1)
    y = lax.bitcast_convert_type(i, jnp.float32)
    vh = 0.5 * v
    for _ in range(2):
        y = y * (1.5 - vh * y * y)
    return y


def _sc_body(ids_hbm, tt_hbm, word_hbm, pos_hbm, type_hbm, gamma_hbm,
             beta_hbm, out_hbm, idx0, idx1, tt0, tt1, rows0, rows1, ob0, ob1,
             comb_v, tv_v, g_v, b_v, gsem0, gsem1, wsem0, wsem1):
    wid = lax.axis_index("s") * 2 + lax.axis_index("c")
    seq0 = wid * _SEQ_PER_W

    pltpu.sync_copy(pos_hbm.at[pl.ds(0, _L)], comb_v.at[pl.ds(0, _L)])
    pltpu.sync_copy(pos_hbm.at[pl.ds(0, _L)], comb_v.at[pl.ds(_L, _L)])
    pltpu.sync_copy(type_hbm, tv_v)
    pltpu.sync_copy(gamma_hbm, g_v)
    pltpu.sync_copy(beta_hbm, b_v)

    sl = [pl.ds(16 * j, 16) for j in range(4)]
    t0 = [tv_v[0, sl[j]] for j in range(4)]
    t1 = [tv_v[1, sl[j]] for j in range(4)]
    gam = [g_v[sl[j]] for j in range(4)]
    bet = [b_v[sl[j]] for j in range(4)]

    @pl.loop(0, _L)
    def _build(r):
        for j in range(4):
            comb_v[r, sl[j]] = comb_v[r, sl[j]] + t0[j]
            comb_v[_L + r, sl[j]] = comb_v[_L + r, sl[j]] + t1[j]

    bufs = ((idx0, tt0, rows0, ob0, gsem0, wsem0),
            (idx1, tt1, rows1, ob1, gsem1, wsem1))

    def wb_copy(c, b):
        ob, wsem = bufs[b][3], bufs[b][5]
        return pltpu.make_async_copy(ob, out_hbm.at[seq0 + c], wsem)

    def issue(c, b):
        idx, tt, rows, _, gsem, _ = bufs[b]
        s = seq0 + c
        for k, off in enumerate(_OFFS):
            pltpu.sync_copy(ids_hbm.at[s, pl.ds(off, 104)], idx.at[k])
        pltpu.sync_copy(tt_hbm.at[s, pl.ds(0, _L)], tt.at[pl.ds(0, _L)])
        for k, off in enumerate(_OFFS):
            pltpu.make_async_copy(word_hbm.at[idx.at[k]],
                                  rows.at[pl.ds(off, 104)], gsem).start()

    def process(c, b):
        idx, tt, rows, ob, gsem, _ = bufs[b]
        for k, off in enumerate(_OFFS):
            pltpu.make_async_copy(word_hbm.at[idx.at[k]],
                                  rows.at[pl.ds(off, 104)], gsem).wait()

        @pl.when(c >= 2)
        def _():
            wb_copy(c - 2, b).wait()

        @pl.loop(0, _L // 8)
        def _group(g):
            rbase = g * 8
            ttv = tt[pl.ds(rbase, 16)]
            for p in range(2):
                xs, stats = [], []
                for i in range(4):
                    r = rbase + 4 * p + i
                    cb = ttv[4 * p + i] * _L + r
                    x = [rows[r, sl[j]] + comb_v[cb, sl[j]] for j in range(4)]
                    s = (x[0] + x[1]) + (x[2] + x[3])
                    sq = ((x[0] * x[0] + x[1] * x[1])
                          + (x[2] * x[2] + x[3] * x[3]))
                    xs.append(x)
                    stats.append((jnp.sum(s), jnp.sum(sq)))
                for i in range(4):
                    r = rbase + 4 * p + i
                    tot, tsq = stats[i]
                    x = xs[i]
                    mean = tot * jnp.float32(1.0 / _H)
                    var = tsq * jnp.float32(1.0 / _H) - mean * mean
                    vv = jnp.full((16,), var + jnp.float32(1e-5), jnp.float32)
                    scale = _rsqrt_newton(vv)
                    mv = jnp.full((16,), mean, jnp.float32)
                    for j in range(4):
                        ob[r, sl[j]] = (x[j] - mv) * scale * gam[j] + bet[j]

        wb_copy(c, b).start()

    issue(0, 0)

    @pl.loop(0, _SEQ_PER_W // 2)
    def _main(i):
        c0 = i * 2
        issue(c0 + 1, 1)
        process(c0, 0)

        @pl.when(c0 + 2 < _SEQ_PER_W)
        def _():
            issue(c0 + 2, 0)

        process(c0 + 1, 1)

    wb_copy(_SEQ_PER_W - 2, 0).wait()
    wb_copy(_SEQ_PER_W - 1, 1).wait()


def kernel(input_ids, token_type_ids, word_emb, pos_emb, type_emb, gamma,
           beta):
    mesh = plsc.VectorSubcoreMesh(core_axis_name="c", subcore_axis_name="s")
    run = functools.partial(
        pl.kernel,
        mesh=mesh,
        compiler_params=pltpu.CompilerParams(
            needs_layout_passes=False, use_tc_tiling_on_sc=False),
        out_type=jax.ShapeDtypeStruct((_B, _L, _H), jnp.float32),
        scratch_types=[
            pltpu.VMEM((2, 104), jnp.int32),
            pltpu.VMEM((2, 104), jnp.int32),
            pltpu.VMEM((208,), jnp.int32),
            pltpu.VMEM((208,), jnp.int32),
            pltpu.VMEM((_L, _H), jnp.float32),
            pltpu.VMEM((_L, _H), jnp.float32),
            pltpu.VMEM((_L, _H), jnp.float32),
            pltpu.VMEM((_L, _H), jnp.float32),
            pltpu.VMEM((2 * _L, _H), jnp.float32),
            pltpu.VMEM((2, _H), jnp.float32),
            pltpu.VMEM((_H,), jnp.float32),
            pltpu.VMEM((_H,), jnp.float32),
            pltpu.SemaphoreType.DMA,
            pltpu.SemaphoreType.DMA,
            pltpu.SemaphoreType.DMA,
            pltpu.SemaphoreType.DMA,
        ],
    )(_sc_body)
    return run(input_ids.astype(jnp.int32), token_type_ids.astype(jnp.int32),
               word_emb, pos_emb, type_emb, gamma, beta)
